# Initial kernel scaffold; baseline (speedup 1.0000x reference)
import functools

import jax
import jax.numpy as jnp
from jax import lax
from jax.experimental import pallas as pl
from jax.experimental.pallas import tpu as pltpu
from jax.experimental.pallas import tpu_sc as plsc

VOCAB = 1000000
EMBED = 128
BATCH = 16384
CTX = 20

NC = 2
NS = 16
L = 16
NW = NC * NS
BPW = BATCH // NW
CH = 16
RPC = CH * CTX
IW = 64
NIDX = RPC // IW
NCHUNK = BPW // CH
NVEC = EMBED // L

_mesh = plsc.VectorSubcoreMesh(core_axis_name="c", subcore_axis_name="s")


@functools.partial(
    pl.kernel,
    mesh=_mesh,
    out_type=jax.ShapeDtypeStruct((BATCH, EMBED), jnp.float32),
    scratch_types=[
        pltpu.VMEM((NIDX, IW), jnp.int32),
        pltpu.VMEM((RPC, EMBED), jnp.float32),
        pltpu.VMEM((CH, EMBED), jnp.float32),
        pltpu.SemaphoreType.DMA,
    ],
)
def _cbow_sc(ctx_hbm, table_hbm, out_hbm, idx_v, rows_v, out_v, sem):
    wid = lax.axis_index("s") * NC + lax.axis_index("c")

    def chunk_body(ck, carry):
        blk = wid * NCHUNK + ck
        pltpu.sync_copy(ctx_hbm.at[pl.ds(blk * NIDX, NIDX)], idx_v)
        cps = [
            pltpu.async_copy(
                table_hbm.at[idx_v.at[j]],
                rows_v.at[pl.ds(j * IW, IW)],
                sem,
            )
            for j in range(NIDX)
        ]
        for cp in cps:
            cp.wait()

        def elem_body(b, carry2):
            r0 = b * CTX
            for c in range(NVEC):
                sl = pl.ds(c * L, L)
                acc = rows_v[r0, sl]
                for p in range(1, CTX):
                    acc = acc + rows_v[r0 + p, sl]
                out_v[b, sl] = acc * (1.0 / CTX)
            return carry2

        lax.fori_loop(0, CH, elem_body, 0)
        pltpu.sync_copy(out_v, out_hbm.at[pl.ds(blk * CH, CH)])
        return carry

    lax.fori_loop(0, NCHUNK, chunk_body, 0)


def kernel(context, table):
    ctx_flat = context.astype(jnp.int32).reshape(-1, IW)
    return _cbow_sc(ctx_flat, table)


# speedup vs baseline: 1.9584x; 1.9584x over previous
import functools

import jax
import jax.numpy as jnp
from jax import lax
from jax.experimental import pallas as pl
from jax.experimental.pallas import tpu as pltpu
from jax.experimental.pallas import tpu_sc as plsc

VOCAB = 1000000
EMBED = 128
BATCH = 16384
CTX = 20

NC = 2
NS = 16
L = 16
NW = NC * NS
BPW = BATCH // NW
CH = 16
RPC = CH * CTX
IW = 64
NIDX = RPC // IW
NCHUNK = BPW // CH
NVEC = EMBED // L

_mesh = plsc.VectorSubcoreMesh(core_axis_name="c", subcore_axis_name="s")


@functools.partial(
    pl.kernel,
    mesh=_mesh,
    out_type=jax.ShapeDtypeStruct((BATCH, EMBED), jnp.float32),
    scratch_types=[
        pltpu.VMEM((NCHUNK * NIDX, IW), jnp.int32),
        pltpu.VMEM((RPC, EMBED), jnp.float32),
        pltpu.VMEM((CH, EMBED), jnp.float32),
        pltpu.SemaphoreType.DMA,
    ],
)
def _cbow_sc(ctx_hbm, table_hbm, out_hbm, idx_v, rows_v, out_v, sem):
    wid = lax.axis_index("s") * NC + lax.axis_index("c")
    pltpu.sync_copy(ctx_hbm.at[wid], idx_v)

    def chunk_body(ck, carry):
        blk = wid * NCHUNK + ck
        cps = [
            pltpu.async_copy(
                table_hbm.at[idx_v.at[ck * NIDX + j]],
                rows_v.at[pl.ds(j * IW, IW)],
                sem,
            )
            for j in range(NIDX)
        ]
        for cp in cps:
            cp.wait()

        def elem_body(b, carry2):
            r0 = b * CTX
            for c in range(NVEC):
                sl = pl.ds(c * L, L)
                acc = rows_v[r0, sl]
                for p in range(1, CTX):
                    acc = acc + rows_v[r0 + p, sl]
                out_v[b, sl] = acc * (1.0 / CTX)
            return carry2

        lax.fori_loop(0, CH, elem_body, 0)
        pltpu.sync_copy(out_v, out_hbm.at[pl.ds(blk * CH, CH)])
        return carry

    lax.fori_loop(0, NCHUNK, chunk_body, 0)


def kernel(context, table):
    ctx_flat = context.astype(jnp.int32).reshape(NW, NCHUNK * NIDX, IW)
    return _cbow_sc(ctx_flat, table)


# speedup vs baseline: 4.6038x; 2.3508x over previous
import functools

import jax
import jax.numpy as jnp
from jax import lax
from jax.experimental import pallas as pl
from jax.experimental.pallas import tpu as pltpu
from jax.experimental.pallas import tpu_sc as plsc

VOCAB = 1000000
EMBED = 128
BATCH = 16384
CTX = 20

NC = 2
NS = 16
L = 16
NW = NC * NS
BPW = BATCH // NW
IW = 64
NCHUNK = BPW // IW
NVEC = EMBED // L

_mesh = plsc.VectorSubcoreMesh(core_axis_name="c", subcore_axis_name="s")


@functools.partial(
    pl.kernel,
    mesh=_mesh,
    out_type=jax.ShapeDtypeStruct((BATCH, EMBED), jnp.float32),
    scratch_types=[
        pltpu.VMEM((CTX, NCHUNK, IW), jnp.int32),
        pltpu.VMEM((IW, EMBED), jnp.float32),
        pltpu.SemaphoreType.DMA,
        pltpu.SemaphoreType.DMA,
    ],
)
def _cbow_sc(ctx_hbm, table_hbm, out_hbm, idx_v, acc_v, sem0, sem1):
    wid = lax.axis_index("s") * NC + lax.axis_index("c")
    pltpu.sync_copy(ctx_hbm.at[wid], idx_v)

    def chunk_body(ck, carry):
        blk = wid * NCHUNK + ck
        pltpu.async_copy(
            table_hbm.at[idx_v.at[0, ck]], acc_v, sem0
        ).wait()
        cps = [
            pltpu.async_copy(
                table_hbm.at[idx_v.at[p, ck]], acc_v, sem1, add=True
            )
            for p in range(1, CTX)
        ]
        for cp in cps:
            cp.wait()

        def elem_body(b, carry2):
            for c in range(NVEC):
                sl = pl.ds(c * L, L)
                acc_v[b, sl] = acc_v[b, sl] * (1.0 / CTX)
            return carry2

        lax.fori_loop(0, IW, elem_body, 0)
        pltpu.sync_copy(acc_v, out_hbm.at[pl.ds(blk * IW, IW)])
        return carry

    lax.fori_loop(0, NCHUNK, chunk_body, 0)


def kernel(context, table):
    ctx_arr = (
        context.astype(jnp.int32)
        .reshape(NW, NCHUNK, IW, CTX)
        .transpose(0, 3, 1, 2)
    )
    return _cbow_sc(ctx_arr, table)


# speedup vs baseline: 5.0601x; 1.0991x over previous
import functools

import jax
import jax.numpy as jnp
from jax import lax
from jax.experimental import pallas as pl
from jax.experimental.pallas import tpu as pltpu
from jax.experimental.pallas import tpu_sc as plsc

VOCAB = 1000000
EMBED = 128
BATCH = 16384
CTX = 20

NC = 2
NS = 16
L = 16
NW = NC * NS
BPW = BATCH // NW
IW = 64
NCHUNK = BPW // IW
NVEC = EMBED // L
NBUF = 2

_mesh = plsc.VectorSubcoreMesh(core_axis_name="c", subcore_axis_name="s")


@functools.partial(
    pl.kernel,
    mesh=_mesh,
    out_type=jax.ShapeDtypeStruct((BATCH, EMBED), jnp.float32),
    scratch_types=[
        pltpu.VMEM((CTX, NCHUNK, IW), jnp.int32),
        pltpu.VMEM((NBUF, IW, EMBED), jnp.float32),
        pltpu.SemaphoreType.DMA,
        pltpu.SemaphoreType.DMA,
    ],
)
def _cbow_sc(ctx_hbm, table_hbm, out_hbm, idx_v, acc_v, sem0, sem1):
    wid = lax.axis_index("s") * NC + lax.axis_index("c")
    sems = [sem0, sem1]
    pltpu.sync_copy(ctx_hbm.at[wid], idx_v)

    def zero_buf(buf):
        def zrow(b, carry):
            for c in range(NVEC):
                acc_v[buf, b, pl.ds(c * L, L)] = jnp.zeros((L,), jnp.float32)
            return carry

        lax.fori_loop(0, IW, zrow, 0)

    def fire(ck, buf, sem):
        return [
            pltpu.async_copy(
                table_hbm.at[idx_v.at[p, ck]],
                acc_v.at[buf],
                sem,
                add=True,
            )
            for p in range(CTX)
        ]

    pending = {}
    for buf in range(NBUF):
        zero_buf(buf)
    for ck in range(NBUF):
        pending[ck] = fire(ck, ck, sems[ck])

    for ck in range(NCHUNK):
        buf = ck % NBUF
        for cp in pending.pop(ck):
            cp.wait()

        def scale_row(b, carry):
            for c in range(NVEC):
                sl = pl.ds(c * L, L)
                acc_v[buf, b, sl] = acc_v[buf, b, sl] * (1.0 / CTX)
            return carry

        lax.fori_loop(0, IW, scale_row, 0)
        pltpu.sync_copy(
            acc_v.at[buf], out_hbm.at[pl.ds((wid * NCHUNK + ck) * IW, IW)]
        )
        if ck + NBUF < NCHUNK:
            zero_buf(buf)
            pending[ck + NBUF] = fire(ck + NBUF, buf, sems[buf])


def kernel(context, table):
    ctx_arr = (
        context.astype(jnp.int32)
        .reshape(NW, NCHUNK, IW, CTX)
        .transpose(0, 3, 1, 2)
    )
    return _cbow_sc(ctx_arr, table)


# speedup vs baseline: 5.2189x; 1.0314x over previous
import functools

import jax
import jax.numpy as jnp
from jax import lax
from jax.experimental import pallas as pl
from jax.experimental.pallas import tpu as pltpu
from jax.experimental.pallas import tpu_sc as plsc

VOCAB = 1000000
EMBED = 128
BATCH = 16384
CTX = 20

NC = 2
NS = 16
L = 16
NW = NC * NS
BPW = BATCH // NW
IW = 128
NCHUNK = BPW // IW
NVEC = EMBED // L
NBUF = 2

_mesh = plsc.VectorSubcoreMesh(core_axis_name="c", subcore_axis_name="s")


@functools.partial(
    pl.kernel,
    mesh=_mesh,
    out_type=jax.ShapeDtypeStruct((BATCH, EMBED), jnp.float32),
    scratch_types=[
        pltpu.VMEM((CTX, NCHUNK, IW), jnp.int32),
        pltpu.VMEM((NBUF, IW, EMBED), jnp.float32),
        pltpu.SemaphoreType.DMA,
        pltpu.SemaphoreType.DMA,
    ],
)
def _cbow_sc(ctx_hbm, table_hbm, out_hbm, idx_v, acc_v, sem0, sem1):
    wid = lax.axis_index("s") * NC + lax.axis_index("c")
    sems = [sem0, sem1]
    pltpu.sync_copy(ctx_hbm.at[wid], idx_v)

    def zero_buf(buf):
        def zrow(b, carry):
            for c in range(NVEC):
                acc_v[buf, b, pl.ds(c * L, L)] = jnp.zeros((L,), jnp.float32)
            return carry

        lax.fori_loop(0, IW, zrow, 0)

    def fire(ck, buf, sem):
        return [
            pltpu.async_copy(
                table_hbm.at[idx_v.at[p, ck]],
                acc_v.at[buf],
                sem,
                add=True,
            )
            for p in range(CTX)
        ]

    pending = {}
    for buf in range(NBUF):
        zero_buf(buf)
    for ck in range(NBUF):
        pending[ck] = fire(ck, ck, sems[ck])

    for ck in range(NCHUNK):
        buf = ck % NBUF
        for cp in pending.pop(ck):
            cp.wait()

        def scale_row(b, carry):
            for c in range(NVEC):
                sl = pl.ds(c * L, L)
                acc_v[buf, b, sl] = acc_v[buf, b, sl] * (1.0 / CTX)
            return carry

        lax.fori_loop(0, IW, scale_row, 0)
        pltpu.sync_copy(
            acc_v.at[buf], out_hbm.at[pl.ds((wid * NCHUNK + ck) * IW, IW)]
        )
        if ck + NBUF < NCHUNK:
            zero_buf(buf)
            pending[ck + NBUF] = fire(ck + NBUF, buf, sems[buf])


def kernel(context, table):
    ctx_arr = (
        context.astype(jnp.int32)
        .reshape(NW, NCHUNK, IW, CTX)
        .transpose(0, 3, 1, 2)
    )
    return _cbow_sc(ctx_arr, table)
